# Initial kernel scaffold; baseline (speedup 1.0000x reference)
import jax
import jax.numpy as jnp
from jax.experimental import pallas as pl


def kernel(seq, feature, labels, weight):
    raise NotImplementedError("write your pallas kernel here")


# speedup vs baseline: 3.4635x; 3.4635x over previous
import functools
import jax
import jax.numpy as jnp
from jax import lax
from jax.experimental import pallas as pl
from jax.experimental.pallas import tpu as pltpu

N = 100000
D = 128
NCLS = 3
CPAD = 8
BLK = 5000
GRID = N // BLK


def _segsum_body(labels_ref, feat_ref, out_ref):
    step = pl.program_id(0)
    lab = labels_ref[0, 0, :]
    cls = lax.broadcasted_iota(jnp.int32, (CPAD, BLK), 0)
    onehot = (cls == lab[None, :]).astype(jnp.float32)
    acc = lax.dot_general(
        onehot, feat_ref[...],
        (((1,), (0,)), ((), ())),
        preferred_element_type=jnp.float32,
    )

    @pl.when(step == 0)
    def _():
        out_ref[...] = acc

    @pl.when(step != 0)
    def _():
        out_ref[...] += acc


def _dense_body(seq_ref, w_ref, seg_ref, out_ref):
    x = seq_ref[...]
    t = x * w_ref[...]
    r = jnp.where(t > 0, t, jnp.exp(jnp.minimum(t, 0.0)) - 1.0)

    seg = seg_ref[...]
    ave = seg * jnp.float32(1.0 / (N // 2))
    an = jnp.sqrt(jnp.sum(ave * ave, axis=1, keepdims=True))
    an = jnp.maximum(an, 1e-8)
    avn = ave / an

    rn = jnp.maximum(jnp.sqrt(jnp.sum(r * r, axis=1, keepdims=True)), 1e-8)
    cos = lax.dot_general(
        r, avn, (((1,), (1,)), ((), ())),
        preferred_element_type=jnp.float32,
    )
    cos = cos / rn
    c3 = cos[:, :NCLS]
    m = jnp.max(c3, axis=1, keepdims=True)
    e = jnp.exp(c3 - m)
    out_ref[...] = e / jnp.sum(e, axis=1, keepdims=True)


def _segment_sum(feature, labels):
    labels3 = labels.reshape(GRID, 1, BLK)
    return pl.pallas_call(
        _segsum_body,
        grid=(GRID,),
        in_specs=[
            pl.BlockSpec((1, 1, BLK), lambda i: (i, 0, 0)),
            pl.BlockSpec((BLK, D), lambda i: (i, 0)),
        ],
        out_specs=pl.BlockSpec((CPAD, D), lambda i: (0, 0)),
        out_shape=jax.ShapeDtypeStruct((CPAD, D), jnp.float32),
    )(labels3, feature)


def _dense(seq, weight, seg):
    return pl.pallas_call(
        _dense_body,
        grid=(GRID,),
        in_specs=[
            pl.BlockSpec((BLK, D), lambda i: (i, 0)),
            pl.BlockSpec((1, D), lambda i: (0, 0)),
            pl.BlockSpec((CPAD, D), lambda i: (0, 0)),
        ],
        out_specs=pl.BlockSpec((BLK, NCLS), lambda i: (i, 0)),
        out_shape=jax.ShapeDtypeStruct((N, NCLS), jnp.float32),
    )(seq, weight, seg)


@jax.jit
def kernel(seq, feature, labels, weight):
    seg = _segment_sum(feature, labels)
    return _dense(seq, weight, seg)


# speedup vs baseline: 6.3778x; 1.8414x over previous
import functools
import jax
import jax.numpy as jnp
from jax import lax
from jax.experimental import pallas as pl
from jax.experimental.pallas import tpu as pltpu

N = 100000
D = 128
NCLS = 3
CPAD = 8
BLK = 5000
GRID = N // BLK


def _segsum_body(labels_ref, feat_ref, out_ref):
    step = pl.program_id(0)
    lab = labels_ref[0, 0, :]
    cls = lax.broadcasted_iota(jnp.int32, (CPAD, BLK), 0)
    onehot = (cls == lab[None, :]).astype(jnp.float32)
    acc = lax.dot_general(
        onehot, feat_ref[...],
        (((1,), (0,)), ((), ())),
        preferred_element_type=jnp.float32,
    )

    @pl.when(step == 0)
    def _():
        out_ref[...] = acc

    @pl.when(step != 0)
    def _():
        out_ref[...] += acc


def _dense_body(seq_ref, w_ref, seg_ref, out_ref):
    x = seq_ref[...]
    t = x * w_ref[...]
    r = jnp.where(t > 0, t, jnp.exp(t) - 1.0)

    seg = seg_ref[...]
    ave = seg * jnp.float32(1.0 / (N // 2))
    an = jnp.sqrt(jnp.sum(ave * ave, axis=1, keepdims=True))
    an = jnp.maximum(an, 1e-8)
    avn = ave / an

    a = lax.dot_general(
        avn, r, (((1,), (1,)), ((), ())),
        preferred_element_type=jnp.float32,
    )
    rr = lax.dot_general(
        jnp.ones((8, D), jnp.float32), r * r, (((1,), (1,)), ((), ())),
        preferred_element_type=jnp.float32,
    )[0:1, :]
    inv_rn = 1.0 / jnp.maximum(jnp.sqrt(rr), 1e-8)
    cos = a * inv_rn

    c0 = cos[0:1, :]
    c1 = cos[1:2, :]
    c2 = cos[2:3, :]
    m = jnp.maximum(jnp.maximum(c0, c1), c2)
    e0 = jnp.exp(c0 - m)
    e1 = jnp.exp(c1 - m)
    e2 = jnp.exp(c2 - m)
    inv_s = 1.0 / (e0 + e1 + e2)
    out_ref[0, 0:1, :] = e0 * inv_s
    out_ref[0, 1:2, :] = e1 * inv_s
    out_ref[0, 2:3, :] = e2 * inv_s


def _segment_sum(feature, labels):
    labels3 = labels.reshape(GRID, 1, BLK)
    return pl.pallas_call(
        _segsum_body,
        grid=(GRID,),
        in_specs=[
            pl.BlockSpec((1, 1, BLK), lambda i: (i, 0, 0)),
            pl.BlockSpec((BLK, D), lambda i: (i, 0)),
        ],
        out_specs=pl.BlockSpec((CPAD, D), lambda i: (0, 0)),
        out_shape=jax.ShapeDtypeStruct((CPAD, D), jnp.float32),
    )(labels3, feature)


def _dense(seq, weight, seg):
    return pl.pallas_call(
        _dense_body,
        grid=(GRID,),
        in_specs=[
            pl.BlockSpec((BLK, D), lambda i: (i, 0)),
            pl.BlockSpec((1, D), lambda i: (0, 0)),
            pl.BlockSpec((CPAD, D), lambda i: (0, 0)),
        ],
        out_specs=pl.BlockSpec((1, NCLS, BLK), lambda i: (i, 0, 0)),
        out_shape=jax.ShapeDtypeStruct((GRID, NCLS, BLK), jnp.float32),
    )(seq, weight, seg)


@jax.jit
def kernel(seq, feature, labels, weight):
    seg = _segment_sum(feature, labels)
    out = _dense(seq, weight, seg)
    return out.transpose(0, 2, 1).reshape(N, NCLS)
